# baseline (speedup 1.0000x reference)
import functools

import jax
import jax.numpy as jnp
from jax import lax
from jax.experimental import pallas as pl
from jax.experimental.pallas import tpu as pltpu
from jax.experimental.pallas import tpu_sc as plsc

IN_CH = 256
OUT_CH = 256
N_NODES = 10000
N_EDGES = 160000

NUM_SC = 2
NUM_TILES = 16
FH = IN_CH // NUM_SC
CA = 144
ROWS_SH = 10112
ZROWS = ROWS_SH // NUM_TILES
CHUNK = 128
E_PAD = 163840
N_CHUNKS = E_PAD // (NUM_TILES * CHUNK)
SB = 4
N_SUPER = N_CHUNKS // SB

_mesh = plsc.VectorSubcoreMesh(core_axis_name="c", subcore_axis_name="s")


@functools.partial(
    pl.kernel,
    out_type=jax.ShapeDtypeStruct((NUM_SC, ROWS_SH, CA), jnp.float32),
    mesh=_mesh,
    scratch_types=[
        pltpu.VMEM((SB, CHUNK), jnp.int32),
        pltpu.VMEM((SB, CHUNK), jnp.int32),
        pltpu.VMEM((CHUNK, CA), jnp.float32),
        pltpu.VMEM((CHUNK, CA), jnp.float32),
        pltpu.VMEM_SHARED((ROWS_SH, CA), jnp.float32),
        pltpu.SemaphoreType.DMA,
        pltpu.SemaphoreType.DMA,
        pltpu.SemaphoreType.DMA,
        pltpu.SemaphoreType.DMA,
    ],
    compiler_params=pltpu.CompilerParams(use_tc_tiling_on_sc=False),
)
def _sc_aggregate(xa_hbm, src_hbm, dst_hbm, zeros_hbm, out_hbm,
                  src_v, dst_v, rows0, rows1, agg_sh,
                  semg0, semg1, sems0, sems1):
    cid = lax.axis_index("c")
    sid = lax.axis_index("s")
    rows = (rows0, rows1)
    semg = (semg0, semg1)
    sems = (sems0, sems1)

    pltpu.sync_copy(zeros_hbm, agg_sh.at[pl.ds(sid * ZROWS, ZROWS)])
    plsc.subcore_barrier()

    def super_body(s, _):
        pltpu.sync_copy(src_hbm.at[sid, pl.ds(s * SB, SB)], src_v)
        pltpu.sync_copy(dst_hbm.at[sid, pl.ds(s * SB, SB)], dst_v)
        gd = {}
        sd = {}

        def gather(j, buf, sem):
            return pltpu.async_copy(
                xa_hbm.at[cid].at[src_v.at[j]], buf, sem)

        gd[0] = gather(0, rows[0], semg[0])
        for j in range(SB):
            b = j & 1
            if j + 1 < SB:
                if j >= 1:
                    sd[j - 1].wait()
                gd[j + 1] = gather(j + 1, rows[1 - b], semg[1 - b])
            gd[j].wait()
            sd[j] = pltpu.async_copy(
                rows[b], agg_sh.at[dst_v.at[j]], sems[b], add=True)
        sd[SB - 2].wait()
        sd[SB - 1].wait()
        return 0

    lax.fori_loop(0, N_SUPER, super_body, 0)
    plsc.subcore_barrier()

    pltpu.sync_copy(agg_sh.at[pl.ds(sid * ZROWS, ZROWS)],
                    out_hbm.at[cid, pl.ds(sid * ZROWS, ZROWS)])


_BR = 400


def _tc_body(x_ref, s_ref, c_ref, w1_ref, w2_ref, b1_ref, b2_ref, o_ref):
    c = c_ref[...]
    inv = 1.0 / jnp.maximum(c, 1.0)
    mean = s_ref[...] * inv
    acc = jnp.dot(x_ref[...], w1_ref[...], preferred_element_type=jnp.float32)
    acc = acc + jnp.dot(mean, w2_ref[...], preferred_element_type=jnp.float32)
    acc = acc + b1_ref[...]
    acc = acc + jnp.where(c > 0.0, 1.0, 0.0) * b2_ref[...]
    o_ref[...] = acc


def _tc_combine(x, sums, cnt, W1, W2, b1, b2):
    return pl.pallas_call(
        _tc_body,
        grid=(N_NODES // _BR,),
        in_specs=[
            pl.BlockSpec((_BR, IN_CH), lambda i: (i, 0)),
            pl.BlockSpec((_BR, IN_CH), lambda i: (i, 0)),
            pl.BlockSpec((_BR, 1), lambda i: (i, 0)),
            pl.BlockSpec((IN_CH, OUT_CH), lambda i: (0, 0)),
            pl.BlockSpec((IN_CH, OUT_CH), lambda i: (0, 0)),
            pl.BlockSpec((1, OUT_CH), lambda i: (0, 0)),
            pl.BlockSpec((1, OUT_CH), lambda i: (0, 0)),
        ],
        out_specs=pl.BlockSpec((_BR, OUT_CH), lambda i: (i, 0)),
        out_shape=jax.ShapeDtypeStruct((N_NODES, OUT_CH), jnp.float32),
    )(x, sums, cnt, W1, W2, b1, b2)


def kernel(x, edge_index, W1, b1, W2, b2):
    src = edge_index[0].astype(jnp.int32)
    dst = edge_index[1].astype(jnp.int32)
    src_p = jnp.pad(src, (0, E_PAD - N_EDGES)).reshape(NUM_TILES, N_CHUNKS, CHUNK)
    dst_p = jnp.pad(dst, (0, E_PAD - N_EDGES),
                    constant_values=N_NODES).reshape(NUM_TILES, N_CHUNKS, CHUNK)
    ones = jnp.ones((N_NODES, 1), jnp.float32)
    zpad = jnp.zeros((N_NODES, CA - FH - 1), jnp.float32)
    xa = jnp.stack(
        [jnp.concatenate([x[:, :FH], ones, zpad], axis=1),
         jnp.concatenate([x[:, FH:], ones, zpad], axis=1)])
    zeros = jnp.zeros((ZROWS, CA), jnp.float32)

    agg = _sc_aggregate(xa, src_p, dst_p, zeros)
    sums = jnp.concatenate(
        [agg[0, :N_NODES, :FH], agg[1, :N_NODES, :FH]], axis=1)
    cnt = agg[0, :N_NODES, FH].reshape(N_NODES, 1)

    return _tc_combine(x, sums, cnt, W1, W2,
                       b1.reshape(1, OUT_CH), b2.reshape(1, OUT_CH))
